# initial kernel scaffold (unmeasured)
import jax
import jax.numpy as jnp
from jax import lax
from jax.experimental import pallas as pl
from jax.experimental.pallas import tpu as pltpu

N_DEV = 16
BLK = 64


def kernel(x, Wq, K_ext, V_ext, Wo):
    B, sq, dm = x.shape
    _, skv, hq, dh = K_ext.shape
    dq = Wq.shape[1]
    nqb = sq // BLK
    nkb = skv // BLK

    K2 = K_ext.reshape(B, skv, hq * dh)
    V2 = V_ext.reshape(B, skv, hq * dh)

    def body(x_ref, wq_ref, k_ref, v_ref, wo_ref, out_ref,
             kc, vc, ksend, krecv, vsend, vrecv):
        my = lax.axis_index("i")
        left = lax.rem(my - 1 + N_DEV, N_DEV)
        right = lax.rem(my + 1, N_DEV)

        barrier_sem = pltpu.get_barrier_semaphore()
        pl.semaphore_signal(barrier_sem, inc=1, device_id=(left,),
                            device_id_type=pl.DeviceIdType.MESH)
        pl.semaphore_signal(barrier_sem, inc=1, device_id=(right,),
                            device_id_type=pl.DeviceIdType.MESH)
        pl.semaphore_wait(barrier_sem, 2)

        kc[0] = k_ref[...]
        vc[0] = v_ref[...]

        for h in range(N_DEV - 1):
            k_rdma = pltpu.make_async_remote_copy(
                src_ref=kc.at[h], dst_ref=kc.at[h + 1],
                send_sem=ksend.at[h], recv_sem=krecv.at[h],
                device_id=(right,), device_id_type=pl.DeviceIdType.MESH)
            v_rdma = pltpu.make_async_remote_copy(
                src_ref=vc.at[h], dst_ref=vc.at[h + 1],
                send_sem=vsend.at[h], recv_sem=vrecv.at[h],
                device_id=(right,), device_id_type=pl.DeviceIdType.MESH)
            k_rdma.start()
            v_rdma.start()
            k_rdma.wait()
            v_rdma.wait()

        qi = lax.broadcasted_iota(jnp.int32, (sq, skv), 0)
        kj = lax.broadcasted_iota(jnp.int32, (sq, skv), 1)
        qb = my * nqb + qi // BLK

        for b in range(B):
            qmat = jnp.dot(x_ref[b], wq_ref[...],
                           preferred_element_type=jnp.float32) * 0.125
            ctx_heads = []
            for hh in range(hq):
                qh = qmat[:, hh * dh:(hh + 1) * dh]
                acc = jnp.zeros((sq, dh), jnp.float32)
                den = jnp.zeros((sq, 1), jnp.float32)
                for s in range(N_DEV):
                    origin = lax.rem(my - s + N_DEV, N_DEV)
                    kcs = kc[s, b]
                    vcs = vc[s, b]
                    ks = kcs[:, hh * dh:(hh + 1) * dh]
                    vs = vcs[:, hh * dh:(hh + 1) * dh]
                    sc = lax.dot_general(
                        qh, ks, (((1,), (1,)), ((), ())),
                        preferred_element_type=jnp.float32)
                    kb = origin * nkb + kj // BLK
                    mask = (qb == kb) | (kb == 0) | (lax.rem(qb + kb, 3) == 0)
                    w = jnp.where(mask, jnp.exp(sc), 0.0)
                    acc = acc + jnp.dot(w, vs,
                                        preferred_element_type=jnp.float32)
                    den = den + jnp.sum(w, axis=1, keepdims=True)
                ctx_heads.append(acc / den)
            ctx = jnp.concatenate(ctx_heads, axis=1)
            out_ref[b] = jnp.dot(ctx, wo_ref[...],
                                 preferred_element_type=jnp.float32)

    return pl.pallas_call(
        body,
        out_shape=jax.ShapeDtypeStruct((B, sq, dm), jnp.float32),
        in_specs=[pl.BlockSpec(memory_space=pltpu.VMEM)] * 5,
        out_specs=pl.BlockSpec(memory_space=pltpu.VMEM),
        scratch_shapes=[
            pltpu.VMEM((N_DEV, B, skv, hq * dh), jnp.float32),
            pltpu.VMEM((N_DEV, B, skv, hq * dh), jnp.float32),
            pltpu.SemaphoreType.DMA((N_DEV - 1,)),
            pltpu.SemaphoreType.DMA((N_DEV - 1,)),
            pltpu.SemaphoreType.DMA((N_DEV - 1,)),
            pltpu.SemaphoreType.DMA((N_DEV - 1,)),
        ],
        compiler_params=pltpu.CompilerParams(collective_id=0),
    )(x, Wq, K2, V2, Wo)


# baseline (device time: 236848 ns/iter reference)
import jax
import jax.numpy as jnp
from jax import lax
from jax.experimental import pallas as pl
from jax.experimental.pallas import tpu as pltpu

N_DEV = 16
BLK = 64


def kernel(x, Wq, K_ext, V_ext, Wo):
    B, sq, dm = x.shape
    _, skv, hq, dh = K_ext.shape
    dq = Wq.shape[1]
    nqb = sq // BLK
    nkb = skv // BLK

    K2 = K_ext.reshape(B, skv, hq * dh)
    V2 = V_ext.reshape(B, skv, hq * dh)

    def body(x_ref, wq_ref, k_ref, v_ref, wo_ref, out_ref,
             kc, vc, ksend, krecv, vsend, vrecv):
        my = lax.axis_index("i")
        left = lax.rem(my - 1 + N_DEV, N_DEV)
        right = lax.rem(my + 1, N_DEV)

        barrier_sem = pltpu.get_barrier_semaphore()
        pl.semaphore_signal(barrier_sem, inc=1, device_id=(left,),
                            device_id_type=pl.DeviceIdType.MESH)
        pl.semaphore_signal(barrier_sem, inc=1, device_id=(right,),
                            device_id_type=pl.DeviceIdType.MESH)
        pl.semaphore_wait(barrier_sem, 2)

        kc[0] = k_ref[...]
        vc[0] = v_ref[...]

        for h in range(N_DEV - 1):
            k_rdma = pltpu.make_async_remote_copy(
                src_ref=kc.at[h], dst_ref=kc.at[h + 1],
                send_sem=ksend.at[h], recv_sem=krecv.at[h],
                device_id=(right,), device_id_type=pl.DeviceIdType.MESH)
            v_rdma = pltpu.make_async_remote_copy(
                src_ref=vc.at[h], dst_ref=vc.at[h + 1],
                send_sem=vsend.at[h], recv_sem=vrecv.at[h],
                device_id=(right,), device_id_type=pl.DeviceIdType.MESH)
            k_rdma.start()
            v_rdma.start()
            k_rdma.wait()
            v_rdma.wait()

        qi = lax.broadcasted_iota(jnp.int32, (sq, skv), 0)
        kj = lax.broadcasted_iota(jnp.int32, (sq, skv), 1)
        qb = my * nqb + qi // BLK

        for b in range(B):
            qmat = jnp.dot(x_ref[b], wq_ref[...],
                           preferred_element_type=jnp.float32) * 0.125
            ctx_heads = []
            for hh in range(hq):
                qh = qmat[:, hh * dh:(hh + 1) * dh]
                acc = jnp.zeros((sq, dh), jnp.float32)
                den = jnp.zeros((sq, 1), jnp.float32)
                for s in range(N_DEV):
                    origin = lax.rem(my - s + N_DEV, N_DEV)
                    kcs = kc[s, b]
                    vcs = vc[s, b]
                    ks = kcs[:, hh * dh:(hh + 1) * dh]
                    vs = vcs[:, hh * dh:(hh + 1) * dh]
                    sc = lax.dot_general(
                        qh, ks, (((1,), (1,)), ((), ())),
                        preferred_element_type=jnp.float32)
                    kb = origin * nkb + kj // BLK
                    mask = (qb == kb) | (kb == 0) | (lax.rem(qb + kb, 3) == 0)
                    w = jnp.where(mask, jnp.exp(sc), 0.0)
                    acc = acc + jnp.dot(w, vs,
                                        preferred_element_type=jnp.float32)
                    den = den + jnp.sum(w, axis=1, keepdims=True)
                ctx_heads.append(acc / den)
            ctx = jnp.concatenate(ctx_heads, axis=1)
            out_ref[b] = jnp.dot(ctx, wo_ref[...],
                                 preferred_element_type=jnp.float32)

    return pl.pallas_call(
        body,
        out_shape=jax.ShapeDtypeStruct((B, sq, dm), jnp.float32),
        in_specs=[pl.BlockSpec(memory_space=pltpu.VMEM)] * 5,
        out_specs=pl.BlockSpec(memory_space=pltpu.VMEM),
        scratch_shapes=[
            pltpu.VMEM((N_DEV, B, skv, hq * dh), jnp.float32),
            pltpu.VMEM((N_DEV, B, skv, hq * dh), jnp.float32),
            pltpu.SemaphoreType.DMA((N_DEV - 1,)),
            pltpu.SemaphoreType.DMA((N_DEV - 1,)),
            pltpu.SemaphoreType.DMA((N_DEV - 1,)),
            pltpu.SemaphoreType.DMA((N_DEV - 1,)),
        ],
        compiler_params=pltpu.CompilerParams(
            collective_id=0, vmem_limit_bytes=64 * 1024 * 1024),
    )(x, Wq, K2, V2, Wo)
